# Initial kernel scaffold; baseline (speedup 1.0000x reference)
import jax
import jax.numpy as jnp
from jax.experimental import pallas as pl


def kernel(img):
    raise NotImplementedError("write your pallas kernel here")


# speedup vs baseline: 304.6770x; 304.6770x over previous
import jax
import jax.numpy as jnp
from jax import lax
from jax.experimental import pallas as pl
from jax.experimental.pallas import tpu as pltpu
from jax.experimental.pallas import tpu_sc as plsc

L = 16
NCH = 96
NPIX = 512 * 512
CHUNK = 65536
NCHUNK = NPIX // CHUNK
NW = 32
CPW = NCH // NW
NBIN = 256


def _body(img_hbm, out_hbm, buf, hist, lut):
    cid = lax.axis_index("c")
    sid = lax.axis_index("s")
    wid = sid * 2 + cid

    ones = jnp.full((L,), 1, jnp.int32)
    zeros = jnp.zeros((L,), jnp.int32)
    iota = lax.iota(jnp.int32, L)
    total = jnp.int32(NPIX)

    for j in range(CPW):
        ch = wid + NW * j

        for k in range(NBIN // L):
            hist[pl.ds(k * L, L)] = zeros
        for c in range(NCHUNK):
            pltpu.sync_copy(img_hbm.at[ch, pl.ds(c * CHUNK, CHUNK)], buf)

            def hist_body(i, _):
                v = buf[pl.ds(i * L, L)]
                plsc.addupdate_scatter(hist, [v], ones)
                return 0

            lax.fori_loop(0, CHUNK // L, hist_body, 0)

        carry = jnp.int32(0)
        m = jnp.int32(0)
        for k in range(NBIN // L):
            h = hist[pl.ds(k * L, L)]
            csum = plsc.cumsum(h) + carry
            carry = jnp.max(csum)
            m = jnp.maximum(m, jnp.max(jnp.where(csum < total, csum, 0)))
            hist[pl.ds(k * L, L)] = csum

        step = lax.div(m, jnp.int32(255))
        half = lax.div(step, jnp.int32(2))
        sstep = jnp.maximum(step, jnp.int32(1))
        is_id = step == 0

        lut[pl.ds(0, L)] = zeros
        for k in range(NBIN // L):
            csum = hist[pl.ds(k * L, L)]
            lv = lax.div(csum + half, sstep)
            lv = jnp.clip(lv, 0, 255)
            idv = iota + (k * L + 1)
            lv = jnp.where(is_id, idv, lv)
            lut[pl.ds(k * L + 1, L)] = lv

        for c in range(NCHUNK):
            pltpu.sync_copy(img_hbm.at[ch, pl.ds(c * CHUNK, CHUNK)], buf)

            def app_body(i, _):
                v = buf[pl.ds(i * L, L)]
                buf[pl.ds(i * L, L)] = plsc.load_gather(lut, [v])
                return 0

            lax.fori_loop(0, CHUNK // L, app_body, 0)
            pltpu.sync_copy(buf, out_hbm.at[ch, pl.ds(c * CHUNK, CHUNK)])


def kernel(img):
    B, C, H, W = img.shape
    flat = img.reshape(NCH, NPIX)
    mesh = plsc.VectorSubcoreMesh(
        core_axis_name="c", subcore_axis_name="s", num_cores=2, num_subcores=16
    )
    out = pl.kernel(
        _body,
        out_type=jax.ShapeDtypeStruct((NCH, NPIX), jnp.int32),
        mesh=mesh,
        scratch_types=[
            pltpu.VMEM((CHUNK,), jnp.int32),
            pltpu.VMEM((NBIN,), jnp.int32),
            pltpu.VMEM((NBIN + L,), jnp.int32),
        ],
        compiler_params=pltpu.CompilerParams(needs_layout_passes=False),
    )(flat)
    return out.reshape(B, C, H, W)


# speedup vs baseline: 720.7582x; 2.3656x over previous
import jax
import jax.numpy as jnp
from jax import lax
from jax.experimental import pallas as pl
from jax.experimental.pallas import tpu as pltpu
from jax.experimental.pallas import tpu_sc as plsc

L = 16
NCH = 96
NPIX = 512 * 512
CHUNK = 65536
NCHUNK = NPIX // CHUNK
NW = 32
CPW = NCH // NW
NBIN = 256


def _body(img_hbm, out_hbm, buf, hist, lut):
    cid = lax.axis_index("c")
    sid = lax.axis_index("s")
    wid = sid * 2 + cid

    ones = jnp.full((L,), 1, jnp.int32)
    zeros = jnp.zeros((L,), jnp.int32)
    iota = lax.iota(jnp.int32, L)
    total = jnp.int32(NPIX)

    for j in range(CPW):
        ch = wid + NW * j

        for k in range(NBIN // L):
            hist[pl.ds(k * L, L)] = zeros
        for c in range(NCHUNK):
            pltpu.sync_copy(img_hbm.at[ch, pl.ds(c * CHUNK, CHUNK)], buf)

            @plsc.parallel_loop(0, CHUNK // L, 1, unroll=8)
            def _(i):
                v = buf[pl.ds(i * L, L)]
                plsc.addupdate_scatter(hist, [v], ones)

        carry = jnp.int32(0)
        m = jnp.int32(0)
        for k in range(NBIN // L):
            h = hist[pl.ds(k * L, L)]
            csum = plsc.cumsum(h) + carry
            carry = jnp.max(csum)
            m = jnp.maximum(m, jnp.max(jnp.where(csum < total, csum, 0)))
            hist[pl.ds(k * L, L)] = csum

        step = lax.div(m, jnp.int32(255))
        half = lax.div(step, jnp.int32(2))
        sstep = jnp.maximum(step, jnp.int32(1))
        is_id = step == 0

        lut[pl.ds(0, L)] = zeros
        for k in range(NBIN // L):
            csum = hist[pl.ds(k * L, L)]
            lv = lax.div(csum + half, sstep)
            lv = jnp.clip(lv, 0, 255)
            idv = iota + (k * L + 1)
            lv = jnp.where(is_id, idv, lv)
            lut[pl.ds(k * L + 1, L)] = lv

        for c in range(NCHUNK):
            pltpu.sync_copy(img_hbm.at[ch, pl.ds(c * CHUNK, CHUNK)], buf)

            @plsc.parallel_loop(0, CHUNK // L, 1, unroll=8)
            def _(i):
                v = buf[pl.ds(i * L, L)]
                buf[pl.ds(i * L, L)] = plsc.load_gather(lut, [v])

            pltpu.sync_copy(buf, out_hbm.at[ch, pl.ds(c * CHUNK, CHUNK)])


def kernel(img):
    B, C, H, W = img.shape
    flat = img.reshape(NCH, NPIX)
    mesh = plsc.VectorSubcoreMesh(
        core_axis_name="c", subcore_axis_name="s", num_cores=2, num_subcores=16
    )
    out = pl.kernel(
        _body,
        out_type=jax.ShapeDtypeStruct((NCH, NPIX), jnp.int32),
        mesh=mesh,
        scratch_types=[
            pltpu.VMEM((CHUNK,), jnp.int32),
            pltpu.VMEM((NBIN,), jnp.int32),
            pltpu.VMEM((NBIN + L,), jnp.int32),
        ],
        compiler_params=pltpu.CompilerParams(needs_layout_passes=False),
    )(flat)
    return out.reshape(B, C, H, W)


# speedup vs baseline: 917.0217x; 1.2723x over previous
import jax
import jax.numpy as jnp
from jax import lax
from jax.experimental import pallas as pl
from jax.experimental.pallas import tpu as pltpu
from jax.experimental.pallas import tpu_sc as plsc

L = 16
NCH = 96
NPIX = 512 * 512
CHUNK = 16384
NCHUNK = NPIX // CHUNK
NBUF = 4
PRE = 2
NW = 32
CPW = NCH // NW
NBIN = 256
UNROLL = 8


def _body(img_hbm, out_hbm, b0, b1, b2, b3, hist, lut, s0, s1, s2, s3):
    bufs = (b0, b1, b2, b3)
    sems = (s0, s1, s2, s3)
    cid = lax.axis_index("c")
    sid = lax.axis_index("s")
    wid = sid * 2 + cid

    ones = jnp.full((L,), 1, jnp.int32)
    zeros = jnp.zeros((L,), jnp.int32)
    iota = lax.iota(jnp.int32, L)
    total = jnp.int32(NPIX)

    def in_dma(ch, c):
        return pltpu.async_copy(
            img_hbm.at[ch, pl.ds(c * CHUNK, CHUNK)], bufs[c % NBUF], sems[c % NBUF]
        )

    def out_dma(ch, c):
        return pltpu.async_copy(
            bufs[c % NBUF], out_hbm.at[ch, pl.ds(c * CHUNK, CHUNK)], sems[c % NBUF]
        )

    def channel_body(j, _):
        ch = wid + NW * j

        for k in range(NBIN // L):
            hist[pl.ds(k * L, L)] = zeros
        pend = {c: in_dma(ch, c) for c in range(PRE)}
        for c in range(NCHUNK):
            n = c + PRE
            if n < NCHUNK:
                pend[n] = in_dma(ch, n)
            pend.pop(c).wait()
            buf = bufs[c % NBUF]

            @plsc.parallel_loop(0, CHUNK // L, 1, unroll=UNROLL)
            def _(i):
                v = buf[pl.ds(i * L, L)]
                plsc.addupdate_scatter(hist, [v], ones)

        pend = {c: in_dma(ch, c) for c in range(PRE)}

        carry = jnp.int32(0)
        m = jnp.int32(0)
        for k in range(NBIN // L):
            h = hist[pl.ds(k * L, L)]
            csum = plsc.cumsum(h) + carry
            carry = jnp.max(csum)
            m = jnp.maximum(m, jnp.max(jnp.where(csum < total, csum, 0)))
            hist[pl.ds(k * L, L)] = csum

        step = lax.div(m, jnp.int32(255))
        half = lax.div(step, jnp.int32(2))
        sstep = jnp.maximum(step, jnp.int32(1))
        is_id = step == 0

        lut[pl.ds(0, L)] = zeros
        for k in range(NBIN // L):
            csum = hist[pl.ds(k * L, L)]
            lv = lax.div(csum + half, sstep)
            lv = jnp.clip(lv, 0, 255)
            idv = iota + (k * L + 1)
            lv = jnp.where(is_id, idv, lv)
            lut[pl.ds(k * L + 1, L)] = lv

        outs = {}
        for c in range(NCHUNK):
            n = c + PRE
            if n < NCHUNK:
                if n >= NBUF:
                    outs.pop(n - NBUF).wait()
                pend[n] = in_dma(ch, n)
            pend.pop(c).wait()
            buf = bufs[c % NBUF]

            @plsc.parallel_loop(0, CHUNK // L, 1, unroll=UNROLL)
            def _(i):
                v = buf[pl.ds(i * L, L)]
                buf[pl.ds(i * L, L)] = plsc.load_gather(lut, [v])

            outs[c] = out_dma(ch, c)
        for c in sorted(outs):
            outs.pop(c).wait()
        return 0

    lax.fori_loop(0, CPW, channel_body, 0)


def kernel(img):
    B, C, H, W = img.shape
    flat = img.reshape(NCH, NPIX)
    mesh = plsc.VectorSubcoreMesh(
        core_axis_name="c", subcore_axis_name="s", num_cores=2, num_subcores=16
    )
    out = pl.kernel(
        _body,
        out_type=jax.ShapeDtypeStruct((NCH, NPIX), jnp.int32),
        mesh=mesh,
        scratch_types=[
            pltpu.VMEM((CHUNK,), jnp.int32),
            pltpu.VMEM((CHUNK,), jnp.int32),
            pltpu.VMEM((CHUNK,), jnp.int32),
            pltpu.VMEM((CHUNK,), jnp.int32),
            pltpu.VMEM((NBIN,), jnp.int32),
            pltpu.VMEM((NBIN + L,), jnp.int32),
            pltpu.SemaphoreType.DMA,
            pltpu.SemaphoreType.DMA,
            pltpu.SemaphoreType.DMA,
            pltpu.SemaphoreType.DMA,
        ],
        compiler_params=pltpu.CompilerParams(needs_layout_passes=False),
    )(flat)
    return out.reshape(B, C, H, W)


# speedup vs baseline: 1994.3894x; 2.1749x over previous
import jax
import jax.numpy as jnp
from jax import lax
from jax.experimental import pallas as pl
from jax.experimental.pallas import tpu as pltpu
from jax.experimental.pallas import tpu_sc as plsc

L = 16
NCH = 96
NPIX = 512 * 512
CHUNK = 16384
NCHUNK = NPIX // CHUNK
NBUF = 4
PRE = 2
NW = 32
CPW = NCH // NW
NBIN = 256
UNROLL = 8


def _body(img_hbm4, out_hbm4, b0, b1, b2, b3, hist, lut, s0, s1, s2, s3):
    img_hbm = img_hbm4.reshape(NCH * 512, 512)
    out_hbm = out_hbm4.reshape(NCH * 512, 512)
    bufs = (b0, b1, b2, b3)
    sems = (s0, s1, s2, s3)
    cid = lax.axis_index("c")
    sid = lax.axis_index("s")
    wid = sid * 2 + cid

    ones = jnp.full((L,), 1, jnp.int32)
    zeros = jnp.zeros((L,), jnp.int32)
    iota = lax.iota(jnp.int32, L)
    total = jnp.int32(NPIX)

    ROWS = CHUNK // 512

    def in_dma(ch, c):
        return pltpu.async_copy(
            img_hbm.at[pl.ds(ch * 512 + c * ROWS, ROWS), :],
            bufs[c % NBUF],
            sems[c % NBUF],
        )

    def out_dma(ch, c):
        return pltpu.async_copy(
            bufs[c % NBUF],
            out_hbm.at[pl.ds(ch * 512 + c * ROWS, ROWS), :],
            sems[c % NBUF],
        )

    def channel_body(j, _):
        ch = wid + NW * j

        for k in range(NBIN // L):
            hist[pl.ds(k * L, L)] = zeros
        pend = {c: in_dma(ch, c) for c in range(PRE)}
        for c in range(NCHUNK):
            n = c + PRE
            if n < NCHUNK:
                pend[n] = in_dma(ch, n)
            pend.pop(c).wait()
            buf = bufs[c % NBUF]

            @plsc.parallel_loop(0, CHUNK // L, 1, unroll=UNROLL)
            def _(i):
                r = lax.shift_right_logical(i, 5)
                col = lax.shift_left(jnp.bitwise_and(i, 31), 4)
                v = buf[r, pl.ds(col, L)]
                plsc.addupdate_scatter(hist, [v], ones)

        pend = {c: in_dma(ch, c) for c in range(PRE)}

        carry = jnp.int32(0)
        m = jnp.int32(0)
        for k in range(NBIN // L):
            h = hist[pl.ds(k * L, L)]
            csum = plsc.cumsum(h) + carry
            carry = jnp.max(csum)
            m = jnp.maximum(m, jnp.max(jnp.where(csum < total, csum, 0)))
            hist[pl.ds(k * L, L)] = csum

        step = lax.div(m, jnp.int32(255))
        half = lax.div(step, jnp.int32(2))
        sstep = jnp.maximum(step, jnp.int32(1))
        is_id = step == 0

        lut[pl.ds(0, L)] = zeros
        for k in range(NBIN // L):
            csum = hist[pl.ds(k * L, L)]
            lv = lax.div(csum + half, sstep)
            lv = jnp.clip(lv, 0, 255)
            idv = iota + (k * L + 1)
            lv = jnp.where(is_id, idv, lv)
            lut[pl.ds(k * L + 1, L)] = lv

        outs = {}
        for c in range(NCHUNK):
            n = c + PRE
            if n < NCHUNK:
                if n >= NBUF:
                    outs.pop(n - NBUF).wait()
                pend[n] = in_dma(ch, n)
            pend.pop(c).wait()
            buf = bufs[c % NBUF]

            @plsc.parallel_loop(0, CHUNK // L, 1, unroll=UNROLL)
            def _(i):
                r = lax.shift_right_logical(i, 5)
                col = lax.shift_left(jnp.bitwise_and(i, 31), 4)
                v = buf[r, pl.ds(col, L)]
                buf[r, pl.ds(col, L)] = plsc.load_gather(lut, [v])

            outs[c] = out_dma(ch, c)
        for c in sorted(outs):
            outs.pop(c).wait()
        return 0

    lax.fori_loop(0, CPW, channel_body, 0)


def kernel(img):
    B, C, H, W = img.shape
    mesh = plsc.VectorSubcoreMesh(
        core_axis_name="c", subcore_axis_name="s", num_cores=2, num_subcores=16
    )
    out = pl.kernel(
        _body,
        out_type=jax.ShapeDtypeStruct((B, C, H, W), jnp.int32),
        mesh=mesh,
        scratch_types=[
            pltpu.VMEM((CHUNK // 512, 512), jnp.int32),
            pltpu.VMEM((CHUNK // 512, 512), jnp.int32),
            pltpu.VMEM((CHUNK // 512, 512), jnp.int32),
            pltpu.VMEM((CHUNK // 512, 512), jnp.int32),
            pltpu.VMEM((NBIN,), jnp.int32),
            pltpu.VMEM((NBIN + L,), jnp.int32),
            pltpu.SemaphoreType.DMA,
            pltpu.SemaphoreType.DMA,
            pltpu.SemaphoreType.DMA,
            pltpu.SemaphoreType.DMA,
        ],
        compiler_params=pltpu.CompilerParams(needs_layout_passes=False),
    )(img)
    return out
